# baseline (speedup 1.0000x reference)
import jax
import jax.numpy as jnp
from jax.experimental import pallas as pl
from jax.experimental.pallas import tpu as pltpu

_BM = 256
_NBUF = 4


def _body(adj_hbm, emb_ref, out_ref, buf, sems):
    c = pl.program_id(0)
    j = pl.program_id(1)
    npc = pl.num_programs(1)

    def start_copy(slot, blk):
        pltpu.make_async_copy(
            adj_hbm.at[pl.ds((c * npc + blk) * _BM, _BM), :],
            buf.at[slot],
            sems.at[slot],
        ).start()

    @pl.when(j == 0)
    def _warmup():
        for s in range(_NBUF - 1):
            start_copy(s, s)

    nxt = j + _NBUF - 1

    @pl.when(nxt < npc)
    def _prefetch():
        start_copy(nxt % _NBUF, nxt)

    slot = j % _NBUF
    pltpu.make_async_copy(
        adj_hbm.at[pl.ds((c * npc + j) * _BM, _BM), :], buf.at[slot],
        sems.at[slot],
    ).wait()
    out_ref[...] = jnp.dot(buf[slot], emb_ref[...],
                           preferred_element_type=jnp.float32)


def kernel(adj, embeds):
    M, K = adj.shape
    _, N = embeds.shape
    npc = M // _BM // 2
    return pl.pallas_call(
        _body,
        grid=(2, npc),
        in_specs=[
            pl.BlockSpec(memory_space=pl.ANY),
            pl.BlockSpec((K, N), lambda c, j: (0, 0)),
        ],
        out_specs=pl.BlockSpec((_BM, N), lambda c, j: (c * npc + j, 0)),
        out_shape=jax.ShapeDtypeStruct((M, N), jnp.float32),
        scratch_shapes=[
            pltpu.VMEM((_NBUF, _BM, K), jnp.float32),
            pltpu.SemaphoreType.DMA((_NBUF,)),
        ],
        compiler_params=pltpu.CompilerParams(
            dimension_semantics=("parallel", "arbitrary"),
        ),
    )(adj, embeds)


# speedup vs baseline: 1.0770x; 1.0770x over previous
import jax
import jax.numpy as jnp
from jax.experimental import pallas as pl
from jax.experimental.pallas import tpu as pltpu

_BM = 512


def _body(adj_ref, emb_ref, out_ref):
    out_ref[...] = jnp.dot(adj_ref[...], emb_ref[...],
                           preferred_element_type=jnp.float32)


def kernel(adj, embeds):
    M, K = adj.shape
    _, N = embeds.shape
    return pl.pallas_call(
        _body,
        grid=(M // _BM,),
        in_specs=[
            pl.BlockSpec((_BM, K), lambda i: (i, 0)),
            pl.BlockSpec((K, N), lambda i: (0, 0)),
        ],
        out_specs=pl.BlockSpec((_BM, N), lambda i: (i, 0)),
        out_shape=jax.ShapeDtypeStruct((M, N), jnp.float32),
        compiler_params=pltpu.CompilerParams(
            dimension_semantics=("parallel",),
        ),
    )(adj, embeds)
